# baseline (speedup 1.0000x reference)
import functools

import jax
import jax.numpy as jnp
import numpy as np
from jax.experimental import pallas as pl
from jax.experimental.pallas import tpu as pltpu

_PROBABILITY = 0.1
_NUM_MODALS = 3
_B, _D = 4096, 1024


def _zero_row_sets():
    rkey = jax.random.key(42)
    k_mask, k_choice = jax.random.split(rkey)
    mask = np.asarray(jax.random.uniform(k_mask, (_B,)) <= _PROBABILITY)
    choice = np.asarray(jax.random.randint(k_choice, (_B,), 0, _NUM_MODALS))
    return [mask & (choice == m) for m in range(_NUM_MODALS)]


def _runs(zero):
    out, s = [], 0
    for i in range(1, len(zero) + 1):
        if i == len(zero) or bool(zero[i]) != bool(zero[s]):
            out.append((s, i - s, bool(zero[s])))
            s = i
    return out


_RUNS = [_runs(z) for z in _zero_row_sets()]
_ZMAX = max(r[1] for runs in _RUNS for r in runs if r[2])


def _body(m0, m1, m2, o0, o1, o2, zbuf, sem):
    zbuf[...] = jnp.zeros_like(zbuf)
    copies = []
    for src, dst, runs in ((m0, o0, _RUNS[0]), (m1, o1, _RUNS[1]), (m2, o2, _RUNS[2])):
        for start, length, is_zero in runs:
            copies.append(pltpu.make_async_copy(
                zbuf.at[pl.ds(0, length)] if is_zero else src.at[pl.ds(start, length)],
                dst.at[pl.ds(start, length)],
                sem,
            ))
    for c in copies:
        c.start()
    for c in copies:
        c.wait()


@jax.jit
def kernel(modal0, modal1, modal2):
    B, D = modal0.shape
    tiles = [m.reshape(B, 8, 128) for m in (modal0, modal1, modal2)]
    any_spec = pl.BlockSpec(memory_space=pl.ANY)
    out = pl.pallas_call(
        _body,
        in_specs=[any_spec] * 3,
        out_specs=[any_spec] * 3,
        out_shape=[jax.ShapeDtypeStruct((B, 8, 128), modal0.dtype)] * 3,
        scratch_shapes=[
            pltpu.VMEM((_ZMAX, 8, 128), jnp.float32),
            pltpu.SemaphoreType.DMA,
        ],
    )(*tiles)
    return tuple(o.reshape(B, D) for o in out)


# speedup vs baseline: 26.4228x; 26.4228x over previous
import functools

import jax
import jax.numpy as jnp
from jax.experimental import pallas as pl

_PROBABILITY = 0.1
_NUM_MODALS = 3
_B, _D = 4096, 1024
_BLK = 512


def _zero_row_masks(B):
    rkey = jax.random.key(42)
    k_mask, k_choice = jax.random.split(rkey)
    mask = jax.random.uniform(k_mask, (B,)) <= _PROBABILITY
    choice = jax.random.randint(k_choice, (B,), 0, _NUM_MODALS)
    return [
        (mask & (choice == m)).astype(jnp.float32)[:, None]
        for m in range(_NUM_MODALS)
    ]


def _body(m0, m1, m2, z0, z1, z2, o0, o1, o2):
    o0[...] = jnp.where(z0[...] != 0, jnp.float32(0), m0[...])
    o1[...] = jnp.where(z1[...] != 0, jnp.float32(0), m1[...])
    o2[...] = jnp.where(z2[...] != 0, jnp.float32(0), m2[...])


@jax.jit
def kernel(modal0, modal1, modal2):
    B, D = modal0.shape
    z0, z1, z2 = _zero_row_masks(B)
    row_spec = pl.BlockSpec((_BLK, D), lambda i: (i, 0))
    msk_spec = pl.BlockSpec((_BLK, 1), lambda i: (i, 0))
    out = pl.pallas_call(
        _body,
        grid=(B // _BLK,),
        in_specs=[row_spec, row_spec, row_spec, msk_spec, msk_spec, msk_spec],
        out_specs=[row_spec, row_spec, row_spec],
        out_shape=[jax.ShapeDtypeStruct((B, D), modal0.dtype)] * 3,
    )(modal0, modal1, modal2, z0, z1, z2)
    return tuple(out)
